# baseline (speedup 1.0000x reference)
import functools
import jax
import jax.numpy as jnp
from jax import lax
from jax.experimental import pallas as pl
from jax.experimental.pallas import tpu as pltpu
from jax.experimental.pallas import tpu_sc as plsc

VOCAB = 1000000
D = 64
B, L = 16384, 20
N = B * L
NC, NS, LANES = 2, 16, 16
NW = NC * NS
PER_W = N // NW
CHUNK = 512
NCHUNK = PER_W // CHUNK
IDX_W = 128
GPC = CHUNK // IDX_W
ROWS_PW = PER_W // IDX_W


@functools.cache
def _build_sc_kernel():
  mesh = plsc.VectorSubcoreMesh(
      core_axis_name="c", subcore_axis_name="s", num_cores=NC, num_subcores=NS)

  @functools.partial(
      pl.kernel,
      out_type=jax.ShapeDtypeStruct((N,), jnp.float32),
      mesh=mesh,
      compiler_params=pltpu.CompilerParams(
          needs_layout_passes=False, use_tc_tiling_on_sc=False),
      scratch_types=[
          pltpu.VMEM((D, D), jnp.float32),
          pltpu.VMEM((D,), jnp.float32),
          pltpu.VMEM((D,), jnp.float32),
          pltpu.VMEM((LANES,), jnp.float32),
          pltpu.VMEM((GPC, IDX_W), jnp.int32),
          pltpu.VMEM((CHUNK, D), jnp.float32),
          pltpu.VMEM((PER_W,), jnp.float32),
          pltpu.SemaphoreType.DMA,
      ],
  )
  def _sc_kernel(table, idxm, wo, ww, bw, bo, out,
                 ww_v, wo_v, bw_v, bo_v, idx_v, rows_v, out_v, sem):
    cid = lax.axis_index("c")
    sid = lax.axis_index("s")
    wid = sid * NC + cid

    pltpu.sync_copy(ww, ww_v)
    pltpu.sync_copy(wo, wo_v)
    pltpu.sync_copy(bw, bw_v)
    pltpu.sync_copy(bo, bo_v)

    wo_r = [wo_v[pl.ds(LANES * k, LANES)] for k in range(D // LANES)]
    accs = [jnp.zeros((LANES,), jnp.float32) for _ in range(D // LANES)]
    for e in range(D):
      we = jnp.take_along_axis(
          wo_r[e // LANES], jnp.full((LANES,), e % LANES, jnp.int32), axis=0,
          mode="promise_in_bounds")
      for j in range(D // LANES):
        accs[j] = accs[j] + we * ww_v[e, pl.ds(LANES * j, LANES)]
    cacc = jnp.zeros((LANES,), jnp.float32)
    for j in range(D // LANES):
      cacc = cacc + bw_v[pl.ds(LANES * j, LANES)] * wo_v[pl.ds(LANES * j, LANES)]
    cvec = bo_v[...] + jnp.sum(cacc)

    base_row = wid * ROWS_PW

    def chunk_body(ci, _):
      pltpu.sync_copy(idxm.at[pl.ds(base_row + ci * GPC, GPC)], idx_v)
      cps = [
          pltpu.async_copy(
              table.at[idx_v.at[j]],
              rows_v.at[pl.ds(j * IDX_W, IDX_W)],
              sem,
          )
          for j in range(GPC)
      ]
      for cp in cps:
        cp.wait()

      def g_body(g, _):
        row0 = g * LANES
        riota = lax.iota(jnp.int32, LANES) + row0
        acc = cvec
        for d in range(D):
          vd = jnp.take_along_axis(
              accs[d // LANES], jnp.full((LANES,), d % LANES, jnp.int32),
              axis=0, mode="promise_in_bounds")
          col = plsc.load_gather(
              rows_v, [riota, jnp.full((LANES,), d, jnp.int32)])
          acc = acc + col * vd
        out_v[pl.ds(ci * CHUNK + row0, LANES)] = acc
        return 0

      lax.fori_loop(0, CHUNK // LANES, g_body, 0)
      return 0

    lax.fori_loop(0, NCHUNK, chunk_body, 0)
    pltpu.sync_copy(out_v, out.at[pl.ds(wid * PER_W, PER_W)])

  return _sc_kernel


@jax.jit
def kernel(feature, lan_emb, W_w, b_w, W_out, b_out):
  idx = feature.astype(jnp.int32).reshape(N // IDX_W, IDX_W)
  wo = W_out.reshape(D)
  bo = jnp.broadcast_to(b_out, (LANES,))
  out_flat = _build_sc_kernel()(lan_emb, idx, wo, W_w, b_w, bo)
  return out_flat.reshape(B, L)


# speedup vs baseline: 1.0425x; 1.0425x over previous
import functools
import jax
import jax.numpy as jnp
from jax import lax
from jax.experimental import pallas as pl
from jax.experimental.pallas import tpu as pltpu
from jax.experimental.pallas import tpu_sc as plsc

VOCAB = 1000000
D = 64
B, L = 16384, 20
N = B * L
NC, NS, LANES = 2, 16, 16
NW = NC * NS
PER_W = N // NW
ROWS_W = B // NW
CHUNK = 512
NCHUNK = PER_W // CHUNK
IDX_W = 128
GPC = CHUNK // IDX_W


@functools.cache
def _build_sc_kernel():
  mesh = plsc.VectorSubcoreMesh(
      core_axis_name="c", subcore_axis_name="s", num_cores=NC, num_subcores=NS)

  @functools.partial(
      pl.kernel,
      out_type=jax.ShapeDtypeStruct((B, L), jnp.float32),
      mesh=mesh,
      compiler_params=pltpu.CompilerParams(
          needs_layout_passes=False, use_tc_tiling_on_sc=False),
      scratch_types=[
          pltpu.VMEM((D, D), jnp.float32),
          pltpu.VMEM((D,), jnp.float32),
          pltpu.VMEM((D,), jnp.float32),
          pltpu.VMEM((LANES,), jnp.float32),
          pltpu.VMEM((ROWS_W, L), jnp.int32),
          pltpu.VMEM((PER_W,), jnp.int32),
          pltpu.VMEM((CHUNK, D), jnp.float32),
          pltpu.VMEM((CHUNK, D), jnp.float32),
          pltpu.VMEM((ROWS_W, L), jnp.float32),
          pltpu.SemaphoreType.DMA,
          pltpu.SemaphoreType.DMA,
      ],
  )
  def _sc_kernel(table, feat, wo, ww, bw, bo, out,
                 ww_v, wo_v, bw_v, bo_v, idx2_v, idx_v, rows0_v, rows1_v,
                 out_v, sem0, sem1):
    cid = lax.axis_index("c")
    sid = lax.axis_index("s")
    wid = sid * NC + cid
    bufs = ((rows0_v, sem0), (rows1_v, sem1))
    iota = lax.iota(jnp.int32, LANES)

    pltpu.sync_copy(ww, ww_v)
    pltpu.sync_copy(wo, wo_v)
    pltpu.sync_copy(bw, bw_v)
    pltpu.sync_copy(bo, bo_v)
    pltpu.sync_copy(feat.at[pl.ds(wid * ROWS_W, ROWS_W), :], idx2_v)

    def flat_body(q, _):
      f = q * LANES + iota
      r = f // L
      c = f - r * L
      idx_v[pl.ds(q * LANES, LANES)] = plsc.load_gather(idx2_v, [r, c])
      return 0

    lax.fori_loop(0, PER_W // LANES, flat_body, 0)

    wo_r = [wo_v[pl.ds(LANES * k, LANES)] for k in range(D // LANES)]
    accs = [jnp.zeros((LANES,), jnp.float32) for _ in range(D // LANES)]
    for e in range(D):
      we = jnp.take_along_axis(
          wo_r[e // LANES], jnp.full((LANES,), e % LANES, jnp.int32), axis=0,
          mode="promise_in_bounds")
      for j in range(D // LANES):
        accs[j] = accs[j] + we * ww_v[e, pl.ds(LANES * j, LANES)]
    cacc = jnp.zeros((LANES,), jnp.float32)
    for j in range(D // LANES):
      cacc = cacc + bw_v[pl.ds(LANES * j, LANES)] * wo_v[pl.ds(LANES * j, LANES)]
    cvec = bo_v[...] + jnp.sum(cacc)

    def fire(ci, buf, sem):
      for k in range(GPC):
        pltpu.async_copy(
            table.at[idx_v.at[pl.ds(ci * CHUNK + k * IDX_W, IDX_W)]],
            buf.at[pl.ds(k * IDX_W, IDX_W)],
            sem,
        )

    def drain(buf, sem):
      for k in range(GPC):
        pltpu.make_async_copy(
            table.at[idx_v.at[pl.ds(k * IDX_W, IDX_W)]],
            buf.at[pl.ds(k * IDX_W, IDX_W)],
            sem,
        ).wait()

    def compute(ci, buf):
      def g_body(g, _):
        row0 = g * LANES
        riota = iota + row0
        acc = cvec
        for d in range(D):
          vd = jnp.take_along_axis(
              accs[d // LANES], jnp.full((LANES,), d % LANES, jnp.int32),
              axis=0, mode="promise_in_bounds")
          col = plsc.load_gather(
              buf, [riota, jnp.full((LANES,), d, jnp.int32)])
          acc = acc + col * vd
        f = ci * CHUNK + riota
        r = f // L
        c = f - r * L
        plsc.store_scatter(out_v, [r, c], acc)
        return 0

      lax.fori_loop(0, CHUNK // LANES, g_body, 0)

    fire(0, *bufs[0])

    def outer(i, _):
      ci = i * 2
      fire(ci + 1, *bufs[1])
      drain(*bufs[0])
      compute(ci, bufs[0][0])

      @pl.when(ci + 2 < NCHUNK)
      def _():
        fire(ci + 2, *bufs[0])

      drain(*bufs[1])
      compute(ci + 1, bufs[1][0])
      return 0

    lax.fori_loop(0, NCHUNK // 2, outer, 0)
    pltpu.sync_copy(out_v, out.at[pl.ds(wid * ROWS_W, ROWS_W), :])

  return _sc_kernel


@jax.jit
def kernel(feature, lan_emb, W_w, b_w, W_out, b_out):
  wo = W_out.reshape(D)
  bo = jnp.broadcast_to(b_out, (LANES,))
  return _build_sc_kernel()(lan_emb, feature.astype(jnp.int32), wo, W_w, b_w, bo)


# speedup vs baseline: 3.5472x; 3.4027x over previous
import functools
import jax
import jax.numpy as jnp
from jax import lax
from jax.experimental import pallas as pl
from jax.experimental.pallas import tpu as pltpu
from jax.experimental.pallas import tpu_sc as plsc

VOCAB = 1000000
D = 64
B, L = 16384, 20
N = B * L
NC, NS, LANES = 2, 16, 16
NW = NC * NS
PER_W = N // NW
ROWS_W = B // NW
IDX_W = 128
GATHERS_W = PER_W // IDX_W
BN = 8192


def _tc_scores(vrow_ref, c_ref, tabT_ref, out_ref):
  res = jax.lax.dot_general(
      vrow_ref[...], tabT_ref[...], (((1,), (0,)), ((), ())),
      precision=jax.lax.Precision.HIGHEST,
      preferred_element_type=jnp.float32)
  out_ref[...] = res + c_ref[0, 0]


@jax.jit
def _scores(lan_emb, W_w, b_w, W_out, b_out):
  tabT = lan_emb.T
  vrow = jnp.dot(W_out, W_w, precision=jax.lax.Precision.HIGHEST)
  c = (jnp.dot(W_out, b_w.reshape(D, 1),
               precision=jax.lax.Precision.HIGHEST) + b_out).reshape(1, 1)
  grid = (VOCAB + BN - 1) // BN
  return pl.pallas_call(
      _tc_scores,
      grid=(grid,),
      in_specs=[
          pl.BlockSpec((1, D), lambda i: (0, 0)),
          pl.BlockSpec((1, 1), lambda i: (0, 0), memory_space=pltpu.SMEM),
          pl.BlockSpec((D, BN), lambda i: (0, i)),
      ],
      out_specs=pl.BlockSpec((1, BN), lambda i: (0, i)),
      out_shape=jax.ShapeDtypeStruct((1, VOCAB), jnp.float32),
  )(vrow, c, tabT)


@functools.cache
def _build_sc_gather():
  mesh = plsc.VectorSubcoreMesh(
      core_axis_name="c", subcore_axis_name="s", num_cores=NC, num_subcores=NS)

  @functools.partial(
      pl.kernel,
      out_type=jax.ShapeDtypeStruct((B, L), jnp.float32),
      mesh=mesh,
      compiler_params=pltpu.CompilerParams(
          needs_layout_passes=False, use_tc_tiling_on_sc=False),
      scratch_types=[
          pltpu.VMEM((ROWS_W, L), jnp.int32),
          pltpu.VMEM((PER_W,), jnp.int32),
          pltpu.VMEM((PER_W,), jnp.float32),
          pltpu.VMEM((ROWS_W, L), jnp.float32),
          pltpu.SemaphoreType.DMA,
      ],
  )
  def _sc_gather(scores, feat, out, idx2_v, idx_v, tmp_v, out2_v, sem):
    cid = lax.axis_index("c")
    sid = lax.axis_index("s")
    wid = sid * NC + cid
    iota = lax.iota(jnp.int32, LANES)

    pltpu.sync_copy(feat.at[pl.ds(wid * ROWS_W, ROWS_W), :], idx2_v)

    def flat_body(q, _):
      f = q * LANES + iota
      r = f // L
      c = f - r * L
      idx_v[pl.ds(q * LANES, LANES)] = plsc.load_gather(idx2_v, [r, c])
      return 0

    lax.fori_loop(0, PER_W // LANES, flat_body, 0)

    for k in range(GATHERS_W):
      pltpu.async_copy(
          scores.at[idx_v.at[pl.ds(k * IDX_W, IDX_W)]],
          tmp_v.at[pl.ds(k * IDX_W, IDX_W)],
          sem,
      )
    for k in range(GATHERS_W):
      pltpu.make_async_copy(
          scores.at[idx_v.at[pl.ds(k * IDX_W, IDX_W)]],
          tmp_v.at[pl.ds(k * IDX_W, IDX_W)],
          sem,
      ).wait()

    def out_body(q, _):
      f = q * LANES + iota
      r = f // L
      c = f - r * L
      plsc.store_scatter(out2_v, [r, c], tmp_v[pl.ds(q * LANES, LANES)])
      return 0

    lax.fori_loop(0, PER_W // LANES, out_body, 0)
    pltpu.sync_copy(out2_v, out.at[pl.ds(wid * ROWS_W, ROWS_W), :])

  return _sc_gather


@jax.jit
def kernel(feature, lan_emb, W_w, b_w, W_out, b_out):
  s = _scores(lan_emb, W_w, b_w, W_out, b_out).reshape(VOCAB)
  return _build_sc_gather()(s, feature.astype(jnp.int32))


# speedup vs baseline: 4.2231x; 1.1905x over previous
import functools
import jax
import jax.numpy as jnp
from jax import lax
from jax.experimental import pallas as pl
from jax.experimental.pallas import tpu as pltpu
from jax.experimental.pallas import tpu_sc as plsc

VOCAB = 1000000
D = 64
B, L = 16384, 20
N = B * L
NC, NS, LANES = 2, 16, 16
NW = NC * NS
PER_W = N // NW
ROWS_W = B // NW
IDX_W = 128
GATHERS_W = PER_W // IDX_W
BN = 8192


def _tc_scores(vrow_ref, c_ref, tabT_ref, out_ref):
  res = jax.lax.dot_general(
      vrow_ref[...], tabT_ref[...], (((1,), (0,)), ((), ())),
      precision=jax.lax.Precision.HIGHEST,
      preferred_element_type=jnp.float32)
  out_ref[...] = res.reshape(BN) + c_ref[0, 0]


@jax.jit
def _scores(lan_emb, W_w, b_w, W_out, b_out):
  tabT = lan_emb.T
  vrow = jnp.dot(W_out, W_w, precision=jax.lax.Precision.HIGHEST)
  c = (jnp.dot(W_out, b_w.reshape(D, 1),
               precision=jax.lax.Precision.HIGHEST) + b_out).reshape(1, 1)
  grid = (VOCAB + BN - 1) // BN
  return pl.pallas_call(
      _tc_scores,
      grid=(grid,),
      in_specs=[
          pl.BlockSpec((1, D), lambda i: (0, 0)),
          pl.BlockSpec((1, 1), lambda i: (0, 0), memory_space=pltpu.SMEM),
          pl.BlockSpec((D, BN), lambda i: (0, i)),
      ],
      out_specs=pl.BlockSpec((BN,), lambda i: (i,)),
      out_shape=jax.ShapeDtypeStruct((VOCAB,), jnp.float32),
  )(vrow, c, tabT)


@functools.cache
def _build_sc_gather():
  mesh = plsc.VectorSubcoreMesh(
      core_axis_name="c", subcore_axis_name="s", num_cores=NC, num_subcores=NS)

  @functools.partial(
      pl.kernel,
      out_type=jax.ShapeDtypeStruct((B, L), jnp.float32),
      mesh=mesh,
      compiler_params=pltpu.CompilerParams(
          needs_layout_passes=False, use_tc_tiling_on_sc=False),
      scratch_types=[
          pltpu.VMEM((ROWS_W, L), jnp.int32),
          pltpu.VMEM((PER_W,), jnp.int32),
          pltpu.VMEM((PER_W,), jnp.float32),
          pltpu.VMEM((ROWS_W, L), jnp.float32),
          pltpu.SemaphoreType.DMA,
      ],
  )
  def _sc_gather(scores, feat, out, idx2_v, idx_v, tmp_v, out2_v, sem):
    cid = lax.axis_index("c")
    sid = lax.axis_index("s")
    wid = sid * NC + cid
    iota = lax.iota(jnp.int32, LANES)

    pltpu.sync_copy(feat.at[pl.ds(wid * ROWS_W, ROWS_W), :], idx2_v)

    def flat_body(q, _):
      f = q * LANES + iota
      r = f // L
      c = f - r * L
      idx_v[pl.ds(q * LANES, LANES)] = plsc.load_gather(idx2_v, [r, c])
      return 0

    lax.fori_loop(0, PER_W // LANES, flat_body, 0)

    for k in range(GATHERS_W):
      pltpu.async_copy(
          scores.at[idx_v.at[pl.ds(k * IDX_W, IDX_W)]],
          tmp_v.at[pl.ds(k * IDX_W, IDX_W)],
          sem,
      )
    for k in range(GATHERS_W):
      pltpu.make_async_copy(
          scores.at[idx_v.at[pl.ds(k * IDX_W, IDX_W)]],
          tmp_v.at[pl.ds(k * IDX_W, IDX_W)],
          sem,
      ).wait()

    def out_body(q, _):
      f = q * LANES + iota
      r = f // L
      c = f - r * L
      plsc.store_scatter(out2_v, [r, c], tmp_v[pl.ds(q * LANES, LANES)])
      return 0

    lax.fori_loop(0, PER_W // LANES, out_body, 0)
    pltpu.sync_copy(out2_v, out.at[pl.ds(wid * ROWS_W, ROWS_W), :])

  return _sc_gather


@jax.jit
def kernel(feature, lan_emb, W_w, b_w, W_out, b_out):
  s = _scores(lan_emb, W_w, b_w, W_out, b_out)
  return _build_sc_gather()(s, feature.astype(jnp.int32))


# speedup vs baseline: 5.2335x; 1.2393x over previous
import functools
import jax
import jax.numpy as jnp
from jax import lax
from jax.experimental import pallas as pl
from jax.experimental.pallas import tpu as pltpu
from jax.experimental.pallas import tpu_sc as plsc

VOCAB = 1000000
D = 64
B, L = 16384, 20
N = B * L
NC, NS, LANES = 2, 16, 16
NW = NC * NS
PER_W = N // NW
ROWS_W = B // NW
IDX_W = 128
GATHERS_W = PER_W // IDX_W
BN = 16384


def _tc_scores(vrow_ref, c_ref, tabT_ref, out_ref):
  res = jax.lax.dot_general(
      vrow_ref[...], tabT_ref[...], (((1,), (0,)), ((), ())),
      precision=jax.lax.Precision.HIGHEST,
      preferred_element_type=jnp.float32)
  out_ref[...] = res.reshape(BN) + c_ref[0, 0]


@jax.jit
def _scores(lan_emb, W_w, b_w, W_out, b_out):
  tabT = lan_emb.T
  vrow = jnp.dot(W_out, W_w, precision=jax.lax.Precision.HIGHEST)
  c = (jnp.dot(W_out, b_w.reshape(D, 1),
               precision=jax.lax.Precision.HIGHEST) + b_out).reshape(1, 1)
  grid = (VOCAB + BN - 1) // BN
  return pl.pallas_call(
      _tc_scores,
      grid=(grid,),
      in_specs=[
          pl.BlockSpec((1, D), lambda i: (0, 0)),
          pl.BlockSpec((1, 1), lambda i: (0, 0), memory_space=pltpu.SMEM),
          pl.BlockSpec((D, BN), lambda i: (0, i)),
      ],
      out_specs=pl.BlockSpec((BN,), lambda i: (i,)),
      out_shape=jax.ShapeDtypeStruct((VOCAB,), jnp.float32),
  )(vrow, c, tabT)


@functools.cache
def _build_sc_gather():
  mesh = plsc.VectorSubcoreMesh(
      core_axis_name="c", subcore_axis_name="s", num_cores=NC, num_subcores=NS)

  @functools.partial(
      pl.kernel,
      out_type=jax.ShapeDtypeStruct((B, L), jnp.float32),
      mesh=mesh,
      compiler_params=pltpu.CompilerParams(
          needs_layout_passes=False, use_tc_tiling_on_sc=False),
      scratch_types=[
          pltpu.VMEM((L, ROWS_W), jnp.int32),
          pltpu.VMEM((PER_W,), jnp.int32),
          pltpu.VMEM((PER_W,), jnp.float32),
          pltpu.VMEM((ROWS_W, L), jnp.float32),
          pltpu.SemaphoreType.DMA,
      ],
  )
  def _sc_gather(scores, featT, out, idx2_v, idx_v, tmp_v, out2_v, sem):
    cid = lax.axis_index("c")
    sid = lax.axis_index("s")
    wid = sid * NC + cid
    iota = lax.iota(jnp.int32, LANES)

    pltpu.sync_copy(featT.at[:, pl.ds(wid * ROWS_W, ROWS_W)], idx2_v)

    def flat_body(q, _):
      f = q * LANES + iota
      r = f // L
      c = f - r * L
      idx_v[pl.ds(q * LANES, LANES)] = plsc.load_gather(idx2_v, [c, r])
      return 0

    lax.fori_loop(0, PER_W // LANES, flat_body, 0)

    for k in range(GATHERS_W):
      pltpu.async_copy(
          scores.at[idx_v.at[pl.ds(k * IDX_W, IDX_W)]],
          tmp_v.at[pl.ds(k * IDX_W, IDX_W)],
          sem,
      )
    for k in range(GATHERS_W):
      pltpu.make_async_copy(
          scores.at[idx_v.at[pl.ds(k * IDX_W, IDX_W)]],
          tmp_v.at[pl.ds(k * IDX_W, IDX_W)],
          sem,
      ).wait()

    def out_body(q, _):
      f = q * LANES + iota
      r = f // L
      c = f - r * L
      plsc.store_scatter(out2_v, [r, c], tmp_v[pl.ds(q * LANES, LANES)])
      return 0

    lax.fori_loop(0, PER_W // LANES, out_body, 0)
    pltpu.sync_copy(out2_v, out.at[pl.ds(wid * ROWS_W, ROWS_W), :])

  return _sc_gather


@jax.jit
def kernel(feature, lan_emb, W_w, b_w, W_out, b_out):
  s = _scores(lan_emb, W_w, b_w, W_out, b_out)
  return _build_sc_gather()(s, feature.astype(jnp.int32).T)


# speedup vs baseline: 6.0232x; 1.1509x over previous
import functools
import jax
import jax.numpy as jnp
from jax import lax
from jax.experimental import pallas as pl
from jax.experimental.pallas import tpu as pltpu
from jax.experimental.pallas import tpu_sc as plsc

VOCAB = 1000000
D = 64
B, L = 16384, 20
N = B * L
NC, NS, LANES = 2, 16, 16
NW = NC * NS
PER_W = N // NW
ROWS_W = B // NW
IDX_W = 128
GATHERS_W = PER_W // IDX_W
BN = 32768


def _tc_scores(vrow_ref, c_ref, tabT_ref, out_ref):
  res = jax.lax.dot_general(
      vrow_ref[...], tabT_ref[...], (((1,), (0,)), ((), ())),
      precision=jax.lax.Precision.HIGHEST,
      preferred_element_type=jnp.float32)
  out_ref[...] = res.reshape(BN) + c_ref[0, 0]


@jax.jit
def _scores(lan_emb, W_w, b_w, W_out, b_out):
  tabT = lan_emb.T
  vrow = jnp.dot(W_out, W_w, precision=jax.lax.Precision.HIGHEST)
  c = (jnp.dot(W_out, b_w.reshape(D, 1),
               precision=jax.lax.Precision.HIGHEST) + b_out).reshape(1, 1)
  grid = (VOCAB + BN - 1) // BN
  return pl.pallas_call(
      _tc_scores,
      grid=(grid,),
      in_specs=[
          pl.BlockSpec((1, D), lambda i: (0, 0)),
          pl.BlockSpec((1, 1), lambda i: (0, 0), memory_space=pltpu.SMEM),
          pl.BlockSpec((D, BN), lambda i: (0, i)),
      ],
      out_specs=pl.BlockSpec((BN,), lambda i: (i,)),
      out_shape=jax.ShapeDtypeStruct((VOCAB,), jnp.float32),
  )(vrow, c, tabT)


@functools.cache
def _build_sc_gather():
  mesh = plsc.VectorSubcoreMesh(
      core_axis_name="c", subcore_axis_name="s", num_cores=NC, num_subcores=NS)

  @functools.partial(
      pl.kernel,
      out_type=jax.ShapeDtypeStruct((L, B), jnp.float32),
      mesh=mesh,
      compiler_params=pltpu.CompilerParams(
          needs_layout_passes=False, use_tc_tiling_on_sc=False),
      scratch_types=[
          pltpu.VMEM((L, ROWS_W), jnp.int32),
          pltpu.VMEM((PER_W,), jnp.int32),
          pltpu.VMEM((PER_W,), jnp.float32),
          pltpu.VMEM((L, ROWS_W), jnp.float32),
          pltpu.SemaphoreType.DMA,
      ],
  )
  def _sc_gather(scores, featT, out, idx2_v, idx_v, tmp_v, out2_v, sem):
    cid = lax.axis_index("c")
    sid = lax.axis_index("s")
    wid = sid * NC + cid
    iota = lax.iota(jnp.int32, LANES)

    pltpu.sync_copy(featT.at[:, pl.ds(wid * ROWS_W, ROWS_W)], idx2_v)

    def flat_body(q, _):
      f = q * LANES + iota
      r = f // L
      c = f - r * L
      idx_v[pl.ds(q * LANES, LANES)] = plsc.load_gather(idx2_v, [c, r])
      return 0

    lax.fori_loop(0, PER_W // LANES, flat_body, 0)

    for k in range(GATHERS_W):
      pltpu.async_copy(
          scores.at[idx_v.at[pl.ds(k * IDX_W, IDX_W)]],
          tmp_v.at[pl.ds(k * IDX_W, IDX_W)],
          sem,
      )
    for k in range(GATHERS_W):
      pltpu.make_async_copy(
          scores.at[idx_v.at[pl.ds(k * IDX_W, IDX_W)]],
          tmp_v.at[pl.ds(k * IDX_W, IDX_W)],
          sem,
      ).wait()

    def out_body(q, _):
      f = q * LANES + iota
      r = f // L
      c = f - r * L
      plsc.store_scatter(out2_v, [c, r], tmp_v[pl.ds(q * LANES, LANES)])
      return 0

    lax.fori_loop(0, PER_W // LANES, out_body, 0)
    pltpu.sync_copy(out2_v, out.at[:, pl.ds(wid * ROWS_W, ROWS_W)])

  return _sc_gather


@jax.jit
def kernel(feature, lan_emb, W_w, b_w, W_out, b_out):
  s = _scores(lan_emb, W_w, b_w, W_out, b_out)
  return _build_sc_gather()(s, feature.astype(jnp.int32).T).T
